# baseline (speedup 1.0000x reference)
import jax
import jax.numpy as jnp
from jax.experimental import pallas as pl
from jax.experimental.pallas import tpu as pltpu

K = 50000
Q = 256
D = 6144
BLK = 400
_EPS = 1e-12
_RTOL = 1e-5
_ATOL = 1e-8


def _count_kernel(data_ref, q_ref, t_ref, out_ref, qT_s, th_s, tol_s):
    @pl.when(pl.program_id(0) == 0)
    def _prep():
        q = q_ref[...]
        t = t_ref[...]
        qn = jnp.clip(jnp.sqrt(jnp.sum(q * q, axis=1, keepdims=True)),
                      _EPS, None)
        tn = jnp.clip(jnp.sqrt(jnp.sum(t * t, axis=1, keepdims=True)),
                      _EPS, None)
        th = jnp.sum(q * t, axis=1, keepdims=True) / (qn * tn)
        tol = _ATOL + _RTOL * jnp.abs(th)
        th_s[...] = th.T
        tol_s[...] = tol.T
        qT_s[...] = q.T / qn.T

    p = jnp.dot(data_ref[...], qT_s[...], preferred_element_type=jnp.float32)
    th = th_s[...]
    mask = jnp.logical_or(p >= th, jnp.abs(p - th) <= tol_s[...])
    partial = jnp.sum(mask.astype(jnp.int32), axis=0, keepdims=True)

    @pl.when(pl.program_id(0) == 0)
    def _():
        out_ref[...] = partial - 1

    @pl.when(pl.program_id(0) != 0)
    def _():
        out_ref[...] += partial


def kernel(queries, truths, data, query_idx_in_rankingset,
           use_actaul_mw_for_retrival, use_jaccard):
    return pl.pallas_call(
        _count_kernel,
        grid=(K // BLK,),
        in_specs=[
            pl.BlockSpec((BLK, D), lambda i: (i, 0)),
            pl.BlockSpec((Q, D), lambda i: (0, 0)),
            pl.BlockSpec((Q, D), lambda i: (0, 0)),
        ],
        out_specs=pl.BlockSpec((1, Q), lambda i: (0, 0)),
        out_shape=jax.ShapeDtypeStruct((1, Q), jnp.int32),
        scratch_shapes=[
            pltpu.VMEM((D, Q), jnp.float32),
            pltpu.VMEM((1, Q), jnp.float32),
            pltpu.VMEM((1, Q), jnp.float32),
        ],
    )(data, queries, truths)


# speedup vs baseline: 1.0077x; 1.0077x over previous
import jax
import jax.numpy as jnp
from jax.experimental import pallas as pl
from jax.experimental.pallas import tpu as pltpu

K = 50000
Q = 256
D = 6144
BLK = 200
STEPS = (K // 2) // BLK
_EPS = 1e-12
_RTOL = 1e-5
_ATOL = 1e-8


def _count_kernel(a_ref, b_ref, q_ref, t_ref, out_ref, qT_s, th_s, tol_s):
    @pl.when(pl.program_id(0) == 0)
    def _prep():
        q = q_ref[...]
        t = t_ref[...]
        qn = jnp.clip(jnp.sqrt(jnp.sum(q * q, axis=1, keepdims=True)),
                      _EPS, None)
        tn = jnp.clip(jnp.sqrt(jnp.sum(t * t, axis=1, keepdims=True)),
                      _EPS, None)
        th = jnp.sum(q * t, axis=1, keepdims=True) / (qn * tn)
        tol = _ATOL + _RTOL * jnp.abs(th)
        th_s[...] = th.T
        tol_s[...] = tol.T
        qT_s[...] = q.T / qn.T

    qT = qT_s[...]
    th = th_s[...]
    tol = tol_s[...]
    partial = jnp.zeros((1, Q), jnp.int32)
    for r in (a_ref, b_ref):
        p = jnp.dot(r[...], qT, preferred_element_type=jnp.float32)
        m = jnp.logical_or(p >= th, jnp.abs(p - th) <= tol)
        partial = partial + jnp.sum(m.astype(jnp.int32), axis=0, keepdims=True)

    @pl.when(pl.program_id(0) == 0)
    def _():
        out_ref[...] = partial - 1

    @pl.when(pl.program_id(0) != 0)
    def _():
        out_ref[...] += partial


def kernel(queries, truths, data, query_idx_in_rankingset,
           use_actaul_mw_for_retrival, use_jaccard):
    return pl.pallas_call(
        _count_kernel,
        grid=(STEPS,),
        in_specs=[
            pl.BlockSpec((BLK, D), lambda i: (i, 0)),
            pl.BlockSpec((BLK, D), lambda i: (i + STEPS, 0)),
            pl.BlockSpec((Q, D), lambda i: (0, 0)),
            pl.BlockSpec((Q, D), lambda i: (0, 0)),
        ],
        out_specs=pl.BlockSpec((1, Q), lambda i: (0, 0)),
        out_shape=jax.ShapeDtypeStruct((1, Q), jnp.int32),
        scratch_shapes=[
            pltpu.VMEM((D, Q), jnp.float32),
            pltpu.VMEM((1, Q), jnp.float32),
            pltpu.VMEM((1, Q), jnp.float32),
        ],
    )(data, data, queries, truths)
